# baseline (speedup 1.0000x reference)
import functools

import jax
import jax.numpy as jnp
from jax import lax
from jax.experimental import pallas as pl
from jax.experimental.pallas import tpu as pltpu
from jax.experimental.pallas import tpu_sc as plsc

_NC = 2
_NS = 16
_NW = _NC * _NS

_CHUNK = 128


@functools.lru_cache(maxsize=None)
def _make_gather(N, D, dtype_name):
    dtype = jnp.dtype(dtype_name)
    n_per_w = N // _NW
    n_chunks = n_per_w // _CHUNK
    mesh = plsc.VectorSubcoreMesh(core_axis_name="c", subcore_axis_name="s")

    def body(idx_hbm, table_hbm, out_hbm, idx_v, rows_v, sem):
        wid = lax.axis_index("s") * _NC + lax.axis_index("c")
        base = wid * n_per_w
        pltpu.sync_copy(idx_hbm.at[wid], idx_v)

        @pl.loop(0, n_chunks)
        def _(j):
            pltpu.async_copy(table_hbm.at[idx_v.at[j]], rows_v, sem).wait()
            pltpu.sync_copy(rows_v, out_hbm.at[pl.ds(base + j * _CHUNK, _CHUNK)])

    return pl.kernel(
        body,
        out_type=jax.ShapeDtypeStruct((N, D), dtype),
        mesh=mesh,
        compiler_params=pltpu.CompilerParams(use_tc_tiling_on_sc=False),
        scratch_types=[
            pltpu.VMEM((n_chunks, _CHUNK), jnp.int32),
            pltpu.VMEM((_CHUNK, D), dtype),
            pltpu.SemaphoreType.DMA,
        ],
    )


def kernel(indices, table):
    B, S = indices.shape
    V, D = table.shape
    N = B * S
    idx = indices.astype(jnp.int32).reshape(_NW, -1, _CHUNK)
    out = _make_gather(N, D, table.dtype.name)(idx, table)
    return out.reshape(B, S, D)


# speedup vs baseline: 1.1128x; 1.1128x over previous
import functools

import jax
import jax.numpy as jnp
from jax import lax
from jax.experimental import pallas as pl
from jax.experimental.pallas import tpu as pltpu
from jax.experimental.pallas import tpu_sc as plsc

_NC = 2
_NS = 16
_NW = _NC * _NS

_CHUNK = 128
_GROUP = 4


@functools.lru_cache(maxsize=None)
def _make_gather(N, D, dtype_name):
    dtype = jnp.dtype(dtype_name)
    n_per_w = N // _NW
    n_chunks = n_per_w // _CHUNK
    mesh = plsc.VectorSubcoreMesh(core_axis_name="c", subcore_axis_name="s")

    K = _GROUP
    n_groups = n_chunks // K

    def body(idx_hbm, table_hbm, out_hbm, idx_v, bufs, gsems, wsems):
        wid = lax.axis_index("s") * _NC + lax.axis_index("c")
        base = wid * n_per_w
        pltpu.sync_copy(idx_hbm.at[wid], idx_v)

        def gather(j, h, b):
            return pltpu.make_async_copy(
                table_hbm.at[idx_v.at[j]], bufs.at[h, b], gsems.at[h])

        def write(j, h, b):
            return pltpu.make_async_copy(
                bufs.at[h, b], out_hbm.at[pl.ds(base + j * _CHUNK, _CHUNK)],
                wsems.at[h])

        @pl.loop(0, n_groups, step=2)
        def _(g0):
            for h in range(2):
                g = g0 + h
                j0 = g * K

                @pl.when(g >= 2)
                def _():
                    for b in range(K):
                        write(0, h, b).wait()

                for b in range(K):
                    gather(j0 + b, h, b).start()
                for b in range(K):
                    gather(j0 + b, h, b).wait()
                for b in range(K):
                    write(j0 + b, h, b).start()

        for h in range(2):
            for b in range(K):
                write(0, h, b).wait()

    return pl.kernel(
        body,
        out_type=jax.ShapeDtypeStruct((N, D), dtype),
        mesh=mesh,
        compiler_params=pltpu.CompilerParams(use_tc_tiling_on_sc=False),
        scratch_types=[
            pltpu.VMEM((n_chunks, _CHUNK), jnp.int32),
            pltpu.VMEM((2, K, _CHUNK, D), dtype),
            pltpu.SemaphoreType.DMA((2,)),
            pltpu.SemaphoreType.DMA((2,)),
        ],
    )


def kernel(indices, table):
    B, S = indices.shape
    V, D = table.shape
    N = B * S
    idx = indices.astype(jnp.int32).reshape(_NW, -1, _CHUNK)
    out = _make_gather(N, D, table.dtype.name)(idx, table)
    return out.reshape(B, S, D)
